# baseline (speedup 1.0000x reference)
import functools

import jax
import jax.numpy as jnp
from jax import lax
from jax.experimental import pallas as pl
from jax.experimental.pallas import tpu as pltpu
from jax.experimental.pallas import tpu_sc as plsc

_H = 32
_W = 32
_D = 768
_NB = 4
_L = 16
_NC = 2
_NS = 16


def _pe_sc_body(row_hbm, col_hbm, out_hbm, row_v, col_v, out_v):
    wid = lax.axis_index("s") * _NC + lax.axis_index("c")
    pltpu.sync_copy(row_hbm.at[wid], row_v)
    pltpu.sync_copy(col_hbm, col_v)

    def w_body(w, carry):
        for ci in range(_D // _L):
            sl = pl.ds(ci * _L, _L)
            out_v[w, sl] = col_v[w, sl] + row_v[sl]
        return carry

    lax.fori_loop(0, _W, w_body, 0)
    pltpu.sync_copy(out_v, out_hbm.at[wid])


_pe_sc = functools.partial(
    pl.kernel,
    out_type=jax.ShapeDtypeStruct((_H, _W, _D), jnp.float32),
    mesh=plsc.VectorSubcoreMesh(core_axis_name="c", subcore_axis_name="s"),
    scratch_types=[
        pltpu.VMEM((_D,), jnp.float32),
        pltpu.VMEM((_W, _D), jnp.float32),
        pltpu.VMEM((_W, _D), jnp.float32),
    ],
)(_pe_sc_body)


def _add_body(x_ref, pe_ref, o_ref):
    o_ref[...] = x_ref[...] + pe_ref[...][None]


def kernel(x, pos_row, pos_col):
    B, SEQ, D = x.shape
    pe = _pe_sc(pos_row, pos_col)
    x4 = x.reshape(B, _H, _W, D)
    out = pl.pallas_call(
        _add_body,
        grid=(B // _NB,),
        in_specs=[
            pl.BlockSpec((_NB, _H, _W, D), lambda b: (b, 0, 0, 0)),
            pl.BlockSpec((_H, _W, D), lambda b: (0, 0, 0)),
        ],
        out_specs=pl.BlockSpec((_NB, _H, _W, D), lambda b: (b, 0, 0, 0)),
        out_shape=jax.ShapeDtypeStruct((B, _H, _W, D), x.dtype),
    )(x4, pe)
    return out.reshape(B, SEQ, D)


# speedup vs baseline: 1.0158x; 1.0158x over previous
import functools

import jax
import jax.numpy as jnp
from jax import lax
from jax.experimental import pallas as pl
from jax.experimental.pallas import tpu as pltpu
from jax.experimental.pallas import tpu_sc as plsc

_H = 32
_W = 32
_D = 768
_SEQ = _H * _W
_NB = 4
_L = 16
_NC = 2
_DC = _D // _L
_SLAB = _W * _D


def _pe_sc_body(row_hbm, colf_hbm, out_hbm, row_v, col_v, out_v):
    wid = lax.axis_index("s") * _NC + lax.axis_index("c")
    pltpu.sync_copy(row_hbm.at[wid], row_v)
    pltpu.sync_copy(colf_hbm, col_v)

    @plsc.parallel_loop(0, _W * _DC, unroll=8)
    def _(i):
        ci = lax.rem(i, _DC)
        out_v[pl.ds(i * _L, _L)] = (
            col_v[pl.ds(i * _L, _L)] + row_v[pl.ds(ci * _L, _L)]
        )

    pltpu.sync_copy(out_v, out_hbm.at[pl.ds(wid * _SLAB, _SLAB)])


_pe_sc = functools.partial(
    pl.kernel,
    out_type=jax.ShapeDtypeStruct((_SEQ * _D,), jnp.float32),
    mesh=plsc.VectorSubcoreMesh(core_axis_name="c", subcore_axis_name="s"),
    scratch_types=[
        pltpu.VMEM((_D,), jnp.float32),
        pltpu.VMEM((_SLAB,), jnp.float32),
        pltpu.VMEM((_SLAB,), jnp.float32),
    ],
)(_pe_sc_body)


def _add_body(x_ref, pe_ref, o_ref):
    o_ref[...] = x_ref[...] + pe_ref[...][None]


def kernel(x, pos_row, pos_col):
    B, SEQ, D = x.shape
    pe = _pe_sc(pos_row, pos_col.reshape(-1)).reshape(SEQ, D)
    out = pl.pallas_call(
        _add_body,
        grid=(B // _NB,),
        in_specs=[
            pl.BlockSpec((_NB, SEQ, D), lambda b: (b, 0, 0)),
            pl.BlockSpec((SEQ, D), lambda b: (0, 0)),
        ],
        out_specs=pl.BlockSpec((_NB, SEQ, D), lambda b: (b, 0, 0)),
        out_shape=jax.ShapeDtypeStruct((B, SEQ, D), x.dtype),
    )(x, pe)
    return out
